# baseline (speedup 1.0000x reference)
import functools

import jax
import jax.numpy as jnp
from jax import lax
from jax.experimental import pallas as pl
from jax.experimental.pallas import tpu as pltpu
from jax.experimental.pallas import tpu_sc as plsc

N_NODES = 10000
N_EDGES = 320000
NC = 2
NS = 16
NW = NC * NS
L = 16

EPW = (N_EDGES // (NW * 128)) * 128
REM = N_EDGES - NW * EPW
C0_UNIT = 1 << 16


def _sc_histograms(edges):
    mesh = plsc.VectorSubcoreMesh(
        core_axis_name="c", subcore_axis_name="s", num_cores=NC, num_subcores=NS
    )

    @functools.partial(
        pl.kernel,
        mesh=mesh,
        compiler_params=pltpu.CompilerParams(
            needs_layout_passes=False, skip_device_barrier=True
        ),
        out_type=jax.ShapeDtypeStruct((NW, N_NODES), jnp.int32),
        scratch_types=[
            pltpu.VMEM((EPW,), jnp.int32),
            pltpu.VMEM((EPW,), jnp.int32),
            pltpu.VMEM((REM,), jnp.int32),
            pltpu.VMEM((REM,), jnp.int32),
            pltpu.VMEM((N_NODES,), jnp.int32),
            pltpu.SemaphoreType.DMA,
        ],
    )
    def hist_kernel(edges_hbm, hist_out, src_v, dst_v, srcr_v, dstr_v, hist_v, sem):
        wid = lax.axis_index("s") * NC + lax.axis_index("c")
        base = wid * EPW
        cp1 = pltpu.async_copy(edges_hbm.at[pl.ds(base, EPW)], src_v, sem)
        cp2 = pltpu.async_copy(
            edges_hbm.at[pl.ds(N_EDGES + base, EPW)], dst_v, sem
        )
        cp3 = pltpu.async_copy(
            edges_hbm.at[pl.ds(NW * EPW, REM)], srcr_v, sem
        )
        cp4 = pltpu.async_copy(
            edges_hbm.at[pl.ds(N_EDGES + NW * EPW, REM)], dstr_v, sem
        )

        zeros = jnp.zeros((L,), jnp.int32)

        def zero_body(i, carry):
            for k in range(4):
                hist_v[pl.ds((i * 4 + k) * L, L)] = zeros
            return carry

        lax.fori_loop(0, N_NODES // (4 * L), zero_body, 0)
        cp1.wait()
        cp2.wait()
        cp3.wait()
        cp4.wait()

        ones = jnp.ones((L,), jnp.int32)
        c0u = jnp.full((L,), C0_UNIT, jnp.int32)

        def edge_body(i, carry):
            s = src_v[pl.ds(i * L, L)]
            d = dst_v[pl.ds(i * L, L)]
            plsc.addupdate_scatter(hist_v, [d], ones)
            plsc.addupdate_scatter(hist_v, [s], c0u, mask=(d == 0))
            return carry

        lax.fori_loop(0, EPW // L, edge_body, 0)

        @pl.when(wid == 0)
        def _():
            def rem_body(i, carry):
                s = srcr_v[pl.ds(i * L, L)]
                d = dstr_v[pl.ds(i * L, L)]
                plsc.addupdate_scatter(hist_v, [d], ones)
                plsc.addupdate_scatter(hist_v, [s], c0u, mask=(d == 0))
                return carry

            lax.fori_loop(0, REM // L, rem_body, 0)

        pltpu.sync_copy(hist_v, hist_out.at[wid])

    return hist_kernel(edges)


def _tc_body(parts_ref, x_ref, wg_ref, bg_ref, wc_ref, bc_ref, out_ref):
    parts = parts_ref[...]
    deg_c = jnp.sum(parts & 0xFFFF, axis=0, keepdims=True)
    c0 = jnp.sum(parts >> 16, axis=0, keepdims=True).astype(jnp.float32)
    deg = deg_c.astype(jnp.float32) + 1.0
    dis = lax.rsqrt(deg)
    col = lax.broadcasted_iota(jnp.int32, (1, N_NODES), 1)
    is0 = col == 0
    dis0 = jnp.sum(jnp.where(is0, dis, 0.0))
    u = c0 * dis + jnp.where(is0, dis0, 0.0)
    s = jnp.dot(u, x_ref[...], preferred_element_type=jnp.float32)
    h = dis0 * jnp.dot(s, wg_ref[...], preferred_element_type=jnp.float32)
    h = h + bg_ref[...]
    out_ref[...] = jnp.dot(h, wc_ref[...], preferred_element_type=jnp.float32) + bc_ref[...]


def _tc_combine(parts, x, W_gcn, b_gcn, W_cls, b_cls):
    return pl.pallas_call(
        _tc_body,
        out_shape=jax.ShapeDtypeStruct((1, W_cls.shape[1]), jnp.float32),
    )(parts, x, W_gcn, b_gcn, W_cls, b_cls)


def kernel(embedding, edges, W_gcn, b_gcn, W_cls, b_cls):
    parts = _sc_histograms(edges.astype(jnp.int32).reshape(-1))
    return _tc_combine(
        parts,
        embedding,
        W_gcn,
        b_gcn.reshape(1, -1),
        W_cls,
        b_cls.reshape(1, -1),
    )


# speedup vs baseline: 1.0788x; 1.0788x over previous
import functools

import jax
import jax.numpy as jnp
from jax import lax
from jax.experimental import pallas as pl
from jax.experimental.pallas import tpu as pltpu
from jax.experimental.pallas import tpu_sc as plsc

N_NODES = 10000
N_EDGES = 320000
NC = 2
NS = 16
NW = NC * NS
L = 16

EPW = (N_EDGES // (NW * 128)) * 128
REM = N_EDGES - NW * EPW
C0_UNIT = 1 << 16
N_PAD = ((N_NODES + 127) // 128) * 128


def _sc_histograms(edges):
    mesh = plsc.VectorSubcoreMesh(
        core_axis_name="c", subcore_axis_name="s", num_cores=NC, num_subcores=NS
    )

    @functools.partial(
        pl.kernel,
        mesh=mesh,
        compiler_params=pltpu.CompilerParams(
            needs_layout_passes=False, skip_device_barrier=True
        ),
        out_type=jax.ShapeDtypeStruct((NW, N_PAD), jnp.int32),
        scratch_types=[
            pltpu.VMEM((2, EPW), jnp.int32),
            pltpu.VMEM((2, REM), jnp.int32),
            pltpu.VMEM((N_PAD,), jnp.int32),
            pltpu.SemaphoreType.DMA,
        ],
    )
    def hist_kernel(edges_hbm, hist_out, ev, evr, hist_v, sem):
        wid = lax.axis_index("s") * NC + lax.axis_index("c")
        off = pl.multiple_of(wid * EPW, 128)
        cp = pltpu.async_copy(edges_hbm.at[:, pl.ds(off, EPW)], ev, sem)
        cpr = pltpu.async_copy(edges_hbm.at[:, pl.ds(NW * EPW, REM)], evr, sem)

        zeros = jnp.zeros((L,), jnp.int32)

        def zero_body(i, carry):
            for k in range(4):
                hist_v[pl.ds((i * 4 + k) * L, L)] = zeros
            return carry

        lax.fori_loop(0, N_PAD // (4 * L), zero_body, 0)
        cp.wait()
        cpr.wait()

        ones = jnp.ones((L,), jnp.int32)
        c0u = jnp.full((L,), C0_UNIT, jnp.int32)

        def make_body(buf):
            def body(i, carry):
                s = buf[0, pl.ds(i * L, L)]
                d = buf[1, pl.ds(i * L, L)]
                plsc.addupdate_scatter(hist_v, [d], ones)
                plsc.addupdate_scatter(hist_v, [s], c0u, mask=(d == 0))
                return carry

            return body

        lax.fori_loop(0, EPW // L, make_body(ev), 0)

        @pl.when(wid == 0)
        def _():
            lax.fori_loop(0, REM // L, make_body(evr), 0)

        pltpu.sync_copy(hist_v, hist_out.at[wid])

    return hist_kernel(edges)


def _tc_body(parts_ref, x_ref, wg_ref, bg_ref, wc_ref, bc_ref, out_ref):
    parts = parts_ref[...]
    deg_c = jnp.sum(parts & 0xFFFF, axis=0, keepdims=True)
    c0 = jnp.sum(parts >> 16, axis=0, keepdims=True).astype(jnp.float32)
    deg = deg_c.astype(jnp.float32) + 1.0
    dis = lax.rsqrt(deg)
    col = lax.broadcasted_iota(jnp.int32, (1, N_PAD), 1)
    is0 = col == 0
    dis0 = jnp.sum(jnp.where(is0, dis, 0.0))
    u = c0 * dis + jnp.where(is0, dis0, 0.0)
    s = jnp.dot(u[:, :N_NODES], x_ref[...], preferred_element_type=jnp.float32)
    h = dis0 * jnp.dot(s, wg_ref[...], preferred_element_type=jnp.float32)
    h = h + bg_ref[...]
    out_ref[...] = jnp.dot(h, wc_ref[...], preferred_element_type=jnp.float32) + bc_ref[...]


def _tc_combine(parts, x, W_gcn, b_gcn, W_cls, b_cls):
    return pl.pallas_call(
        _tc_body,
        out_shape=jax.ShapeDtypeStruct((1, W_cls.shape[1]), jnp.float32),
    )(parts, x, W_gcn, b_gcn, W_cls, b_cls)


def kernel(embedding, edges, W_gcn, b_gcn, W_cls, b_cls):
    parts = _sc_histograms(edges.astype(jnp.int32))
    return _tc_combine(
        parts,
        embedding,
        W_gcn,
        b_gcn.reshape(1, -1),
        W_cls,
        b_cls.reshape(1, -1),
    )
